# baseline (speedup 1.0000x reference)
import jax
import jax.numpy as jnp
from jax.experimental import pallas as pl


_ROWS = 150528
_LANES = 128
_GRID = 21
_BLOCK_ROWS = _ROWS // _GRID


def _copy_kernel(x_ref, o_ref):
    o_ref[...] = x_ref[...]


def kernel(feature_batch, box_batch):
    x = feature_batch.reshape(_ROWS, _LANES)
    out = pl.pallas_call(
        _copy_kernel,
        grid=(_GRID,),
        in_specs=[pl.BlockSpec((_BLOCK_ROWS, _LANES), lambda i: (i, 0))],
        out_specs=pl.BlockSpec((_BLOCK_ROWS, _LANES), lambda i: (i, 0)),
        out_shape=jax.ShapeDtypeStruct((_ROWS, _LANES), jnp.float32),
    )(x)
    return out.reshape(feature_batch.shape)
